# baseline (speedup 1.0000x reference)
import functools

import jax
import jax.numpy as jnp
from jax import lax
from jax.experimental import pallas as pl
from jax.experimental.pallas import tpu as pltpu
from jax.experimental.pallas import tpu_sc as plsc

N_ATOMS = 100000
N_EDGES = 6400000
N_MOL = 512

NC = 2
NS = 16
NW = NC * NS
E_PER = N_EDGES // NW
BLK = 1600
NBLK = E_PER // BLK
NGRP = BLK // 16
A_PAD = 100096
A_PER = A_PAD // NS
NCOMP = 6
PAIRS = ((0, 0), (1, 1), (2, 2), (0, 1), (0, 2), (1, 2))
SAC = N_MOL * 8
SRED = NCOMP * N_MOL
SSEG = SRED // NS


def _body(rx, ry, rz, ii_h, ij_h, im_h,
          fx_o, fy_o, fz_o, s_o,
          molv,
          rxv0, ryv0, rzv0, iiv0, ijv0, pxv0, pyv0, pzv0, nxv0, nyv0, nzv0,
          rxv1, ryv1, rzv1, iiv1, ijv1, pxv1, pyv1, pzv1, nxv1, nyv1, nzv1,
          sacc, sred, res,
          fx_s, fy_s, fz_s, im_s, slab, semg, semsc, semin):
    c = lax.axis_index("c")
    s = lax.axis_index("s")
    w = c * NS + s
    lane = lax.iota(jnp.int32, 16)
    zf = jnp.zeros((16,), jnp.float32)

    chunks = []
    off = 0
    while off < A_PER:
        chunks.append((off, min(SRED, A_PER - off)))
        off += SRED

    def zloop(i, _):
        sred[pl.ds(i * 16, 16)] = zf
        return 0
    lax.fori_loop(0, SRED // 16, zloop, 0)
    a0 = s * A_PER
    for f_s in (fx_s, fy_s, fz_s):
        for o, n in chunks:
            pltpu.sync_copy(sred.at[pl.ds(0, n)], f_s.at[pl.ds(a0 + o, n)])

    for k in range(3):
        pltpu.sync_copy(im_h.at[pl.ds(a0 + k * BLK, BLK)], iiv0)
        pltpu.sync_copy(iiv0, im_s.at[pl.ds(a0 + k * BLK, BLK)])
    IREM = A_PER - 3 * BLK
    pltpu.sync_copy(im_h.at[pl.ds(a0 + 3 * BLK, IREM)],
                    iiv0.at[pl.ds(0, IREM)])
    pltpu.sync_copy(iiv0.at[pl.ds(0, IREM)], im_s.at[pl.ds(a0 + 3 * BLK, IREM)])

    def zacc(i, _):
        for t in range(NCOMP):
            sacc[pl.ds(i * 16 + t * SAC, 16)] = zf
        return 0
    lax.fori_loop(0, SAC // 16, zacc, 0)

    plsc.subcore_barrier()

    bufs0 = (rxv0, ryv0, rzv0, iiv0, ijv0, pxv0, pyv0, pzv0, nxv0, nyv0, nzv0)
    bufs1 = (rxv1, ryv1, rzv1, iiv1, ijv1, pxv1, pyv1, pzv1, nxv1, nyv1, nzv1)

    def fire_in(b, bufs):
        base = w * E_PER + b * BLK
        rxv, ryv, rzv, iiv, ijv = bufs[:5]
        pltpu.async_copy(rx.at[pl.ds(base, BLK)], rxv, semin)
        pltpu.async_copy(ry.at[pl.ds(base, BLK)], ryv, semin)
        pltpu.async_copy(rz.at[pl.ds(base, BLK)], rzv, semin)
        pltpu.async_copy(ii_h.at[pl.ds(base, BLK)], iiv, semin)
        pltpu.async_copy(ij_h.at[pl.ds(base, BLK)], ijv, semin)

    def wait_in(bufs):
        rxv, ryv, rzv, iiv, ijv = bufs[:5]
        pltpu.make_async_copy(rx.at[pl.ds(0, BLK)], rxv, semin).wait()
        pltpu.make_async_copy(ry.at[pl.ds(0, BLK)], ryv, semin).wait()
        pltpu.make_async_copy(rz.at[pl.ds(0, BLK)], rzv, semin).wait()
        pltpu.make_async_copy(ii_h.at[pl.ds(0, BLK)], iiv, semin).wait()
        pltpu.make_async_copy(ij_h.at[pl.ds(0, BLK)], ijv, semin).wait()

    def drain_sc():
        for _ in range(6):
            pltpu.make_async_copy(rx.at[pl.ds(0, BLK)], rxv0, semsc).wait()

    def half(b, first, last, bufs, nbufs):
        rxv, ryv, rzv, iiv, ijv, pxv, pyv, pzv, nxv, nyv, nzv = bufs
        wait_in(bufs)
        if not first:
            drain_sc()
        if not last:
            fire_in(b + 1, nbufs)
        gd = pltpu.async_copy(im_s.at[iiv], molv, semg)

        def grp1(j, _):
            sl = pl.ds(j * 16, 16)
            ax = rxv[sl]
            ay = ryv[sl]
            az = rzv[sl]
            r2 = ax * ax + ay * ay + az * az
            ew = jnp.exp(-0.5 * r2)
            px = ax * ew
            py = ay * ew
            pz = az * ew
            pxv[sl] = px
            pyv[sl] = py
            pzv[sl] = pz
            nxv[sl] = -px
            nyv[sl] = -py
            nzv[sl] = -pz
            return 0
        lax.fori_loop(0, NGRP, grp1, 0)

        pltpu.async_copy(nxv, fx_s.at[iiv], semsc, add=True)
        pltpu.async_copy(nyv, fy_s.at[iiv], semsc, add=True)
        pltpu.async_copy(nzv, fz_s.at[iiv], semsc, add=True)
        pltpu.async_copy(pxv, fx_s.at[ijv], semsc, add=True)
        pltpu.async_copy(pyv, fy_s.at[ijv], semsc, add=True)
        pltpu.async_copy(pzv, fz_s.at[ijv], semsc, add=True)

        gd.wait()

        def grp2(j, _):
            sl = pl.ds(j * 16, 16)
            ax = rxv[sl]
            ay = ryv[sl]
            az = rzv[sl]
            nx = nxv[sl]
            ny = nyv[sl]
            nz = nzv[sl]
            mg = molv[sl]
            gb = mg * 8 + jnp.bitwise_and(lane, 7)
            mlo = lane < 8
            mhi = lane >= 8
            nb = (nx, ny, nz)
            ra = (ax, ay, az)
            for comp, (a, bb) in enumerate(PAIRS):
                v = nb[bb] * ra[a]
                plsc.addupdate_scatter(
                    sacc, [gb + comp * SAC], v, mask=mlo)
                plsc.addupdate_scatter(
                    sacc, [gb + comp * SAC], v, mask=mhi)
            return 0
        lax.fori_loop(0, NGRP, grp2, 0)

    fire_in(0, bufs0)

    def blk2(i, _):
        b0 = i * 2

        @pl.when(i == 0)
        def _():
            half(b0, True, False, bufs0, bufs1)

        @pl.when(i > 0)
        def _():
            half(b0, False, False, bufs0, bufs1)

        half(b0 + 1, False, False, bufs1, bufs0)
        return 0
    lax.fori_loop(0, NBLK // 2, blk2, 0)

    half(NBLK - 1, False, True, bufs0, bufs1)

    drain_sc()

    plsc.subcore_barrier()

    for f_s, f_o in ((fx_s, fx_o), (fy_s, fy_o), (fz_s, fz_o)):
        for o, n in chunks:
            pltpu.sync_copy(f_s.at[pl.ds(a0 + o, n)], sred.at[pl.ds(0, n)])
            pltpu.sync_copy(sred.at[pl.ds(0, n)],
                            f_o.at[pl.ds(c * A_PAD + a0 + o, n)])

    for comp in range(NCOMP):
        def lred(m, _, comp=comp):
            ib = m * 128 + lane * 8
            acc = zf
            for l in range(8):
                acc = acc + plsc.load_gather(sacc, [ib + (l + comp * SAC)])
            sred[pl.ds(comp * N_MOL + m * 16, 16)] = acc
            return 0
        lax.fori_loop(0, N_MOL // 16, lred, 0)
    pltpu.sync_copy(sred, slab.at[pl.ds(s * SRED, SRED)])

    plsc.subcore_barrier()

    for t in range(NS):
        pltpu.sync_copy(slab.at[pl.ds(t * SRED + s * SSEG, SSEG)],
                        sred.at[pl.ds(t * SSEG, SSEG)])

    def tred(j, _):
        acc = zf
        for t in range(NS):
            acc = acc + sred[pl.ds(t * SSEG + j * 16, 16)]
        res[pl.ds(j * 16, 16)] = acc
        return 0
    lax.fori_loop(0, SSEG // 16, tred, 0)
    pltpu.sync_copy(res, s_o.at[pl.ds(c * SRED + s * SSEG, SSEG)])


@jax.jit
def kernel(energy, position, Rij, idx_i, idx_j, idx_m, cell):
    rt = Rij.T
    rx = rt[0]
    ry = rt[1]
    rz = rt[2]
    im_pad = jnp.concatenate(
        [idx_m, jnp.zeros((A_PAD - N_ATOMS,), jnp.int32)])

    mesh = plsc.VectorSubcoreMesh(core_axis_name="c", subcore_axis_name="s")
    run = functools.partial(
        pl.kernel,
        out_type=[
            jax.ShapeDtypeStruct((NC * A_PAD,), jnp.float32),
            jax.ShapeDtypeStruct((NC * A_PAD,), jnp.float32),
            jax.ShapeDtypeStruct((NC * A_PAD,), jnp.float32),
            jax.ShapeDtypeStruct((NC * SRED,), jnp.float32),
        ],
        mesh=mesh,
        compiler_params=pltpu.CompilerParams(needs_layout_passes=False),
        scratch_types=(
            [pltpu.VMEM((BLK,), jnp.int32)]
            + 2 * ([pltpu.VMEM((BLK,), jnp.float32)] * 3
                   + [pltpu.VMEM((BLK,), jnp.int32)] * 2
                   + [pltpu.VMEM((BLK,), jnp.float32)] * 6
                   )
            + [
                pltpu.VMEM((NCOMP * SAC,), jnp.float32),
                pltpu.VMEM((SRED,), jnp.float32),
                pltpu.VMEM((SSEG,), jnp.float32),
                pltpu.VMEM_SHARED((A_PAD,), jnp.float32),
                pltpu.VMEM_SHARED((A_PAD,), jnp.float32),
                pltpu.VMEM_SHARED((A_PAD,), jnp.float32),
                pltpu.VMEM_SHARED((A_PAD,), jnp.int32),
                pltpu.VMEM_SHARED((NS * SRED,), jnp.float32),
                pltpu.SemaphoreType.DMA,
                pltpu.SemaphoreType.DMA,
                pltpu.SemaphoreType.DMA,
            ]
        ),
    )(_body)
    fx, fy, fz, s_par = run(rx, ry, rz, idx_i, idx_j, im_pad)

    fx = fx.reshape(NC, A_PAD)
    fy = fy.reshape(NC, A_PAD)
    fz = fz.reshape(NC, A_PAD)
    Fx = (fx[0] + fx[1])[:N_ATOMS]
    Fy = (fy[0] + fy[1])[:N_ATOMS]
    Fz = (fz[0] + fz[1])[:N_ATOMS]
    Fpred = jnp.stack([Fx, Fy, Fz], axis=-1)

    s_par = s_par.reshape(NC, SRED)
    s6 = (s_par[0] + s_par[1]).reshape(NCOMP, N_MOL)
    s00, s11, s22, s01, s02, s12 = s6
    stress = jnp.stack([
        jnp.stack([s00, s01, s02]),
        jnp.stack([s01, s11, s12]),
        jnp.stack([s02, s12, s22]),
    ]).transpose(2, 0, 1)
    cell_33 = cell.reshape(N_MOL, 3, 3)
    volume = jnp.sum(
        cell_33[:, 0, :] * jnp.cross(cell_33[:, 1, :], cell_33[:, 2, :]),
        axis=1, keepdims=True)
    volume = jnp.broadcast_to(volume, (N_MOL, 3)).reshape(N_MOL * 3, 1)
    stress_out = stress.reshape(N_MOL * 3, 3) / volume
    return Fpred, stress_out


# speedup vs baseline: 1.1663x; 1.1663x over previous
import functools

import jax
import jax.numpy as jnp
from jax import lax
from jax.experimental import pallas as pl
from jax.experimental.pallas import tpu as pltpu
from jax.experimental.pallas import tpu_sc as plsc

N_ATOMS = 100000
N_EDGES = 6400000
N_MOL = 512

NC = 2
NS = 16
NW = NC * NS
E_PER = N_EDGES // NW
BLK = 1600
NBLK = E_PER // BLK
NGRP = BLK // 16
A_PAD = 100096
A_PER = A_PAD // NS
NCOMP = 9
SAC = N_MOL * 8
SRED = NCOMP * N_MOL
SSEG = SRED // NS


def _body(rx, ry, rz, ii_h, ij_h, im_h,
          fx_o, fy_o, fz_o, s_o,
          molv,
          rxv0, ryv0, rzv0, iiv0, ijv0, pxv0, pyv0, pzv0, nxv0, nyv0, nzv0,
          rxv1, ryv1, rzv1, iiv1, ijv1, pxv1, pyv1, pzv1, nxv1, nyv1, nzv1,
          sacc, sred, res,
          fx_s, fy_s, fz_s, im_s, slab, semg, semsc, semin):
    c = lax.axis_index("c")
    s = lax.axis_index("s")
    w = c * NS + s
    lane = lax.iota(jnp.int32, 16)
    zf = jnp.zeros((16,), jnp.float32)

    chunks = []
    off = 0
    while off < A_PER:
        chunks.append((off, min(SRED, A_PER - off)))
        off += SRED

    def zloop(i, _):
        sred[pl.ds(i * 16, 16)] = zf
        return 0
    lax.fori_loop(0, SRED // 16, zloop, 0)
    a0 = s * A_PER
    for f_s in (fx_s, fy_s, fz_s):
        for o, n in chunks:
            pltpu.sync_copy(sred.at[pl.ds(0, n)], f_s.at[pl.ds(a0 + o, n)])

    for k in range(3):
        pltpu.sync_copy(im_h.at[pl.ds(a0 + k * BLK, BLK)], iiv0)
        pltpu.sync_copy(iiv0, im_s.at[pl.ds(a0 + k * BLK, BLK)])
    IREM = A_PER - 3 * BLK
    pltpu.sync_copy(im_h.at[pl.ds(a0 + 3 * BLK, IREM)],
                    iiv0.at[pl.ds(0, IREM)])
    pltpu.sync_copy(iiv0.at[pl.ds(0, IREM)], im_s.at[pl.ds(a0 + 3 * BLK, IREM)])

    def zacc(i, _):
        for t in range(NCOMP):
            sacc[pl.ds(i * 16 + t * SAC, 16)] = zf
        return 0
    lax.fori_loop(0, SAC // 16, zacc, 0)

    plsc.subcore_barrier()

    bufs0 = (rxv0, ryv0, rzv0, iiv0, ijv0, pxv0, pyv0, pzv0, nxv0, nyv0, nzv0)
    bufs1 = (rxv1, ryv1, rzv1, iiv1, ijv1, pxv1, pyv1, pzv1, nxv1, nyv1, nzv1)

    def fire_in(b, bufs):
        base = w * E_PER + b * BLK
        rxv, ryv, rzv, iiv, ijv = bufs[:5]
        pltpu.async_copy(rx.at[pl.ds(base, BLK)], rxv, semin)
        pltpu.async_copy(ry.at[pl.ds(base, BLK)], ryv, semin)
        pltpu.async_copy(rz.at[pl.ds(base, BLK)], rzv, semin)
        pltpu.async_copy(ii_h.at[pl.ds(base, BLK)], iiv, semin)
        pltpu.async_copy(ij_h.at[pl.ds(base, BLK)], ijv, semin)

    def wait_in(bufs):
        rxv, ryv, rzv, iiv, ijv = bufs[:5]
        pltpu.make_async_copy(rx.at[pl.ds(0, BLK)], rxv, semin).wait()
        pltpu.make_async_copy(ry.at[pl.ds(0, BLK)], ryv, semin).wait()
        pltpu.make_async_copy(rz.at[pl.ds(0, BLK)], rzv, semin).wait()
        pltpu.make_async_copy(ii_h.at[pl.ds(0, BLK)], iiv, semin).wait()
        pltpu.make_async_copy(ij_h.at[pl.ds(0, BLK)], ijv, semin).wait()

    def drain_sc():
        for _ in range(6):
            pltpu.make_async_copy(rx.at[pl.ds(0, BLK)], rxv0, semsc).wait()

    def half(b, first, last, bufs, nbufs):
        rxv, ryv, rzv, iiv, ijv, pxv, pyv, pzv, nxv, nyv, nzv = bufs
        wait_in(bufs)
        if not first:
            drain_sc()
        if not last:
            fire_in(b + 1, nbufs)
        gd = pltpu.async_copy(im_s.at[iiv], molv, semg)

        def grp1(j, _):
            sl = pl.ds(j * 16, 16)
            ax = rxv[sl]
            ay = ryv[sl]
            az = rzv[sl]
            r2 = ax * ax + ay * ay + az * az
            ew = jnp.exp(-0.5 * r2)
            px = ax * ew
            py = ay * ew
            pz = az * ew
            pxv[sl] = px
            pyv[sl] = py
            pzv[sl] = pz
            nxv[sl] = -px
            nyv[sl] = -py
            nzv[sl] = -pz
            return 0
        lax.fori_loop(0, NGRP, grp1, 0)

        pltpu.async_copy(nxv, fx_s.at[iiv], semsc, add=True)
        pltpu.async_copy(nyv, fy_s.at[iiv], semsc, add=True)
        pltpu.async_copy(nzv, fz_s.at[iiv], semsc, add=True)
        pltpu.async_copy(pxv, fx_s.at[ijv], semsc, add=True)
        pltpu.async_copy(pyv, fy_s.at[ijv], semsc, add=True)
        pltpu.async_copy(pzv, fz_s.at[ijv], semsc, add=True)

        gd.wait()

        def grp2(j, _):
            sl = pl.ds(j * 16, 16)
            ax = rxv[sl]
            ay = ryv[sl]
            az = rzv[sl]
            nx = nxv[sl]
            ny = nyv[sl]
            nz = nzv[sl]
            mg = molv[sl]
            gb = mg * 8 + jnp.bitwise_and(lane, 7)
            mlo = lane < 8
            mhi = lane >= 8
            nb = (nx, ny, nz)
            ra = (ax, ay, az)
            for a in range(3):
                for bb in range(3):
                    comp = 3 * a + bb
                    v = nb[bb] * ra[a]
                    plsc.addupdate_scatter(
                        sacc, [gb + comp * SAC], v, mask=mlo)
                    plsc.addupdate_scatter(
                        sacc, [gb + comp * SAC], v, mask=mhi)
            return 0
        lax.fori_loop(0, NGRP, grp2, 0)

    fire_in(0, bufs0)

    def blk2(i, _):
        b0 = i * 2

        @pl.when(i == 0)
        def _():
            half(b0, True, False, bufs0, bufs1)

        @pl.when(i > 0)
        def _():
            half(b0, False, False, bufs0, bufs1)

        half(b0 + 1, False, False, bufs1, bufs0)
        return 0
    lax.fori_loop(0, NBLK // 2, blk2, 0)

    half(NBLK - 1, False, True, bufs0, bufs1)

    drain_sc()

    plsc.subcore_barrier()

    for f_s, f_o in ((fx_s, fx_o), (fy_s, fy_o), (fz_s, fz_o)):
        for o, n in chunks:
            pltpu.sync_copy(f_s.at[pl.ds(a0 + o, n)], sred.at[pl.ds(0, n)])
            pltpu.sync_copy(sred.at[pl.ds(0, n)],
                            f_o.at[pl.ds(c * A_PAD + a0 + o, n)])

    for comp in range(NCOMP):
        def lred(m, _, comp=comp):
            ib = m * 128 + lane * 8
            acc = zf
            for l in range(8):
                acc = acc + plsc.load_gather(sacc, [ib + (l + comp * SAC)])
            sred[pl.ds(comp * N_MOL + m * 16, 16)] = acc
            return 0
        lax.fori_loop(0, N_MOL // 16, lred, 0)
    pltpu.sync_copy(sred, slab.at[pl.ds(s * SRED, SRED)])

    plsc.subcore_barrier()

    for t in range(NS):
        pltpu.sync_copy(slab.at[pl.ds(t * SRED + s * SSEG, SSEG)],
                        sred.at[pl.ds(t * SSEG, SSEG)])

    def tred(j, _):
        acc = zf
        for t in range(NS):
            acc = acc + sred[pl.ds(t * SSEG + j * 16, 16)]
        res[pl.ds(j * 16, 16)] = acc
        return 0
    lax.fori_loop(0, SSEG // 16, tred, 0)
    pltpu.sync_copy(res, s_o.at[pl.ds(c * SRED + s * SSEG, SSEG)])


@jax.jit
def kernel(energy, position, Rij, idx_i, idx_j, idx_m, cell):
    rt = Rij.T
    rx = rt[0]
    ry = rt[1]
    rz = rt[2]
    im_pad = jnp.concatenate(
        [idx_m, jnp.zeros((A_PAD - N_ATOMS,), jnp.int32)])

    mesh = plsc.VectorSubcoreMesh(core_axis_name="c", subcore_axis_name="s")
    run = functools.partial(
        pl.kernel,
        out_type=[
            jax.ShapeDtypeStruct((NC * A_PAD,), jnp.float32),
            jax.ShapeDtypeStruct((NC * A_PAD,), jnp.float32),
            jax.ShapeDtypeStruct((NC * A_PAD,), jnp.float32),
            jax.ShapeDtypeStruct((NC * SRED,), jnp.float32),
        ],
        mesh=mesh,
        compiler_params=pltpu.CompilerParams(needs_layout_passes=False),
        scratch_types=(
            [pltpu.VMEM((BLK,), jnp.int32)]
            + 2 * ([pltpu.VMEM((BLK,), jnp.float32)] * 3
                   + [pltpu.VMEM((BLK,), jnp.int32)] * 2
                   + [pltpu.VMEM((BLK,), jnp.float32)] * 6
                   )
            + [
                pltpu.VMEM((NCOMP * SAC,), jnp.float32),
                pltpu.VMEM((SRED,), jnp.float32),
                pltpu.VMEM((SSEG,), jnp.float32),
                pltpu.VMEM_SHARED((A_PAD,), jnp.float32),
                pltpu.VMEM_SHARED((A_PAD,), jnp.float32),
                pltpu.VMEM_SHARED((A_PAD,), jnp.float32),
                pltpu.VMEM_SHARED((A_PAD,), jnp.int32),
                pltpu.VMEM_SHARED((NS * SRED,), jnp.float32),
                pltpu.SemaphoreType.DMA,
                pltpu.SemaphoreType.DMA,
                pltpu.SemaphoreType.DMA,
            ]
        ),
    )(_body)
    fx, fy, fz, s_par = run(rx, ry, rz, idx_i, idx_j, im_pad)

    fx = fx.reshape(NC, A_PAD)
    fy = fy.reshape(NC, A_PAD)
    fz = fz.reshape(NC, A_PAD)
    Fx = (fx[0] + fx[1])[:N_ATOMS]
    Fy = (fy[0] + fy[1])[:N_ATOMS]
    Fz = (fz[0] + fz[1])[:N_ATOMS]
    Fpred = jnp.stack([Fx, Fy, Fz], axis=-1)

    s_par = s_par.reshape(NC, SRED)
    stress = (s_par[0] + s_par[1]).reshape(3, 3, N_MOL).transpose(2, 0, 1)
    cell_33 = cell.reshape(N_MOL, 3, 3)
    volume = jnp.sum(
        cell_33[:, 0, :] * jnp.cross(cell_33[:, 1, :], cell_33[:, 2, :]),
        axis=1, keepdims=True)
    volume = jnp.broadcast_to(volume, (N_MOL, 3)).reshape(N_MOL * 3, 1)
    stress_out = stress.reshape(N_MOL * 3, 3) / volume
    return Fpred, stress_out
